# baseline (speedup 1.0000x reference)
import functools

import jax
import jax.numpy as jnp
from jax import lax
from jax.experimental import pallas as pl
from jax.experimental.pallas import tpu as pltpu
from jax.experimental.pallas import tpu_sc as plsc

NUM_OPERATIONS = 1000000
EMBED_DIM = 64
BATCH = 16384

NC = 2
NS = 16
NW = NC * NS
B_PER_W = BATCH // NW
CHUNK = 128
NCHUNK = B_PER_W // CHUNK
LANES = 16
NSLICE = B_PER_W // LANES


def _rsqrt_newton(x):
    i = lax.bitcast_convert_type(x, jnp.int32)
    i = jnp.int32(0x5F3759DF) - (i >> 1)
    y = lax.bitcast_convert_type(i, jnp.float32)
    for _ in range(3):
        y = y * (1.5 - 0.5 * x * y * y)
    return y


def _sc_body(tab_hbm, idx_hbm, out_hbm, idx_v, cols_v, sem):
    wid = lax.axis_index("s") * NC + lax.axis_index("c")
    base = wid * B_PER_W

    pltpu.sync_copy(idx_hbm.at[pl.ds(base, B_PER_W)], idx_v)

    def fire(c, carry):
        row = tab_hbm.at[c]
        for j in range(NCHUNK):
            pltpu.async_copy(
                row.at[idx_v.at[pl.ds(j * CHUNK, CHUNK)]],
                cols_v.at[c, pl.ds(j * CHUNK, CHUNK)],
                sem,
            )
        return carry

    lax.fori_loop(0, EMBED_DIM, fire, 0)

    pltpu.make_async_copy(
        out_hbm.at[:, pl.ds(base, B_PER_W)], cols_v, sem
    ).wait()

    def norm_body(s, carry):
        col = pl.ds(s * LANES, LANES)
        acc = jnp.zeros((LANES,), jnp.float32)
        for c in range(EMBED_DIM):
            x = cols_v[c, col]
            acc = acc + x * x
        acc = jnp.maximum(acc, jnp.float32(1e-30))
        inv = jnp.minimum(_rsqrt_newton(acc), jnp.float32(1e12))
        for c in range(EMBED_DIM):
            cols_v[c, col] = cols_v[c, col] * inv
        return carry

    lax.fori_loop(0, NSLICE, norm_body, 0)

    pltpu.sync_copy(cols_v, out_hbm.at[:, pl.ds(base, B_PER_W)])


@functools.lru_cache(maxsize=None)
def _build():
    mesh = plsc.VectorSubcoreMesh(
        core_axis_name="c", subcore_axis_name="s", num_cores=NC, num_subcores=NS
    )
    return pl.kernel(
        _sc_body,
        out_type=jax.ShapeDtypeStruct((EMBED_DIM, BATCH), jnp.float32),
        mesh=mesh,
        scratch_types=[
            pltpu.VMEM((B_PER_W,), jnp.int32),
            pltpu.VMEM((EMBED_DIM, B_PER_W), jnp.float32),
            pltpu.SemaphoreType.DMA,
        ],
        compiler_params=pltpu.CompilerParams(use_tc_tiling_on_sc=False),
    )


def kernel(operation_ids, table):
    idx = operation_ids.astype(jnp.int32)
    out_t = _build()(table.T, idx)
    return out_t.T


# speedup vs baseline: 7.8279x; 7.8279x over previous
import functools

import jax
import jax.numpy as jnp
from jax import lax
from jax.experimental import pallas as pl
from jax.experimental.pallas import tpu as pltpu
from jax.experimental.pallas import tpu_sc as plsc

NUM_OPERATIONS = 1000000
EMBED_DIM = 64
BATCH = 16384

NC = 2
NS = 16
NW = NC * NS
B_PER_W = BATCH // NW
CHUNK = 128
NCHUNK = B_PER_W // CHUNK
LANES = 16
NSLICE = B_PER_W // LANES


def _rsqrt_newton(x):
    i = lax.bitcast_convert_type(x, jnp.int32)
    i = jnp.int32(0x5F3759DF) - (i >> 1)
    y = lax.bitcast_convert_type(i, jnp.float32)
    for _ in range(3):
        y = y * (1.5 - 0.5 * x * y * y)
    return y


def _sc_body(tab_hbm, idx_hbm, out_hbm, idx_v, rows_v, cols_v, sem):
    wid = lax.axis_index("s") * NC + lax.axis_index("c")
    base = wid * B_PER_W

    pltpu.sync_copy(idx_hbm.at[pl.ds(base, B_PER_W)], idx_v)

    copies = [
        pltpu.async_copy(
            tab_hbm.at[idx_v.at[pl.ds(j * CHUNK, CHUNK)]],
            rows_v.at[pl.ds(j * CHUNK, CHUNK)],
            sem,
        )
        for j in range(NCHUNK)
    ]
    for c in copies:
        c.wait()

    lanes16 = lax.iota(jnp.int32, LANES)

    def norm_body(s, carry):
        rows16 = lanes16 + s * LANES
        col = pl.ds(s * LANES, LANES)
        acc = jnp.zeros((LANES,), jnp.float32)
        for c in range(EMBED_DIM):
            x = plsc.load_gather(
                rows_v, [rows16, jnp.full((LANES,), c, jnp.int32)]
            )
            acc = acc + x * x
            cols_v[c, col] = x
        acc = jnp.maximum(acc, jnp.float32(1e-30))
        inv = jnp.minimum(_rsqrt_newton(acc), jnp.float32(1e12))
        for c in range(EMBED_DIM):
            cols_v[c, col] = cols_v[c, col] * inv
        return carry

    lax.fori_loop(0, NSLICE, norm_body, 0)

    pltpu.sync_copy(cols_v, out_hbm.at[:, pl.ds(base, B_PER_W)])


@functools.lru_cache(maxsize=None)
def _build():
    mesh = plsc.VectorSubcoreMesh(
        core_axis_name="c", subcore_axis_name="s", num_cores=NC, num_subcores=NS
    )
    return pl.kernel(
        _sc_body,
        out_type=jax.ShapeDtypeStruct((EMBED_DIM, BATCH), jnp.float32),
        mesh=mesh,
        scratch_types=[
            pltpu.VMEM((B_PER_W,), jnp.int32),
            pltpu.VMEM((B_PER_W, EMBED_DIM), jnp.float32),
            pltpu.VMEM((EMBED_DIM, B_PER_W), jnp.float32),
            pltpu.SemaphoreType.DMA,
        ],
        compiler_params=pltpu.CompilerParams(use_tc_tiling_on_sc=False, needs_layout_passes=False),
    )


def kernel(operation_ids, table):
    idx = operation_ids.astype(jnp.int32)
    out_t = _build()(table, idx)
    return out_t.T


# speedup vs baseline: 7.8319x; 1.0005x over previous
import functools

import jax
import jax.numpy as jnp
from jax import lax
from jax.experimental import pallas as pl
from jax.experimental.pallas import tpu as pltpu
from jax.experimental.pallas import tpu_sc as plsc

NUM_OPERATIONS = 1000000
EMBED_DIM = 64
BATCH = 16384

NC = 2
NS = 16
NW = NC * NS
B_PER_W = BATCH // NW
CHUNK = 128
NCHUNK = B_PER_W // CHUNK
LANES = 16
NSLICE = B_PER_W // LANES


def _rsqrt_newton(x):
    i = lax.bitcast_convert_type(x, jnp.int32)
    i = jnp.int32(0x5F3759DF) - (i >> 1)
    y = lax.bitcast_convert_type(i, jnp.float32)
    for _ in range(3):
        y = y * (1.5 - 0.5 * x * y * y)
    return y


def _sc_body(tab_hbm, idx_hbm, out_hbm, idx_v, rows_v, cols_v, sem):
    wid = lax.axis_index("s") * NC + lax.axis_index("c")
    base = wid * B_PER_W

    pltpu.sync_copy(idx_hbm.at[pl.ds(base, B_PER_W)], idx_v)

    copies = [
        pltpu.async_copy(
            tab_hbm.at[idx_v.at[pl.ds(j * CHUNK, CHUNK)]],
            rows_v.at[pl.ds(j * CHUNK, CHUNK)],
            sem,
        )
        for j in range(NCHUNK)
    ]
    for c in copies:
        c.wait()

    lanes16 = lax.iota(jnp.int32, LANES)

    def norm_body(s, carry):
        rows16 = lanes16 + s * LANES
        col = pl.ds(s * LANES, LANES)
        acc = jnp.zeros((LANES,), jnp.float32)
        for c in range(EMBED_DIM):
            x = plsc.load_gather(
                rows_v, [rows16, jnp.full((LANES,), c, jnp.int32)]
            )
            acc = acc + x * x
            cols_v[c, col] = x
        acc = jnp.maximum(acc, jnp.float32(1e-30))
        inv = jnp.minimum(_rsqrt_newton(acc), jnp.float32(1e12))
        for c in range(EMBED_DIM):
            cols_v[c, col] = cols_v[c, col] * inv
        return carry

    lax.fori_loop(0, NSLICE, norm_body, 0)

    pltpu.sync_copy(cols_v, out_hbm.at[:, pl.ds(base, B_PER_W)])


@functools.lru_cache(maxsize=None)
def _build():
    mesh = plsc.VectorSubcoreMesh(
        core_axis_name="c", subcore_axis_name="s", num_cores=NC, num_subcores=NS
    )
    return pl.kernel(
        _sc_body,
        out_type=jax.ShapeDtypeStruct((EMBED_DIM, BATCH), jnp.float32),
        mesh=mesh,
        scratch_types=[
            pltpu.VMEM((B_PER_W,), jnp.int32),
            pltpu.VMEM((B_PER_W, EMBED_DIM), jnp.float32),
            pltpu.VMEM((EMBED_DIM, B_PER_W), jnp.float32),
            pltpu.SemaphoreType.DMA,
        ],
        compiler_params=pltpu.CompilerParams(use_tc_tiling_on_sc=False, needs_layout_passes=False, skip_device_barrier=True),
    )


def kernel(operation_ids, table):
    idx = operation_ids.astype(jnp.int32)
    out_t = _build()(table, idx)
    return out_t.T
